# baseline (speedup 1.0000x reference)
import jax
import jax.numpy as jnp
from jax import lax
from jax.experimental import pallas as pl
from jax.experimental.pallas import tpu as pltpu
from jax.experimental.pallas import tpu_sc as plsc

_N, _E, _D = 10000, 320000, 128
_NC, _NS = 2, 16
_NW = _NC * _NS
_CH = 128
_CPB = 16
_NB = 5
_EPT = _NB * _CPB * _CH
_EPAD = _NW * _EPT
_ACC_ROWS = _N + 16
_STRIPE = 624
_REM_OFF = _STRIPE * _NS
_REM = _N - _REM_OFF


def _sc_scatter_body(x_hbm, ei_hbm, zero_hbm, out_hbm,
                     blk_a, blk_b, rows, acc_sh, gsem, ssem):
    c = lax.axis_index("c")
    s = lax.axis_index("s")
    wid = c * _NS + s

    pltpu.sync_copy(zero_hbm.at[pl.ds(s * _STRIPE, _STRIPE)],
                    acc_sh.at[pl.ds(s * _STRIPE, _STRIPE)])

    @pl.when(s == _NS - 1)
    def _():
        pltpu.sync_copy(zero_hbm.at[pl.ds(_REM_OFF, _REM)],
                        acc_sh.at[pl.ds(_REM_OFF, _REM)])

    pltpu.sync_copy(ei_hbm.at[wid, 0], blk_a)
    pltpu.async_copy(ei_hbm.at[wid, 1], blk_b, ssem)
    pltpu.async_copy(x_hbm.at[blk_a.at[0, 0]], rows.at[0], gsem)
    plsc.subcore_barrier()

    def wait_gather(buf):
        pltpu.make_async_copy(x_hbm.at[blk_a.at[0, 0]], buf, gsem).wait()

    def process_block(blk, nxt):
        def pair(i, carry):
            c0 = 2 * i
            c1 = c0 + 1
            wait_gather(rows.at[0])
            pltpu.async_copy(x_hbm.at[blk.at[c1, 0]], rows.at[1], gsem)
            pltpu.sync_copy(rows.at[0], acc_sh.at[blk.at[c0, 1]], add=True)
            wait_gather(rows.at[1])
            pltpu.async_copy(x_hbm.at[blk.at[c0 + 2, 0]], rows.at[0], gsem)
            pltpu.sync_copy(rows.at[1], acc_sh.at[blk.at[c1, 1]], add=True)
            return carry

        lax.fori_loop(0, _CPB // 2 - 1, pair, 0)
        c0 = _CPB - 2
        c1 = _CPB - 1
        wait_gather(rows.at[0])
        pltpu.async_copy(x_hbm.at[blk.at[c1, 0]], rows.at[1], gsem)
        pltpu.sync_copy(rows.at[0], acc_sh.at[blk.at[c0, 1]], add=True)
        wait_gather(rows.at[1])
        if nxt is not None:
            pltpu.make_async_copy(ei_hbm.at[0, 0], nxt, ssem).wait()
            pltpu.async_copy(x_hbm.at[nxt.at[0, 0]], rows.at[0], gsem)
        pltpu.sync_copy(rows.at[1], acc_sh.at[blk.at[c1, 1]], add=True)

    for b in range(_NB):
        blk, nxt = (blk_a, blk_b) if b % 2 == 0 else (blk_b, blk_a)
        process_block(blk, nxt if b < _NB - 1 else None)
        if b + 2 < _NB:
            pltpu.async_copy(ei_hbm.at[wid, b + 2], blk, ssem)

    plsc.subcore_barrier()
    pltpu.sync_copy(acc_sh.at[pl.ds(s * _STRIPE, _STRIPE)],
                    out_hbm.at[pl.ds(c * _N + s * _STRIPE, _STRIPE)])

    @pl.when(s == _NS - 1)
    def _():
        pltpu.sync_copy(acc_sh.at[pl.ds(_REM_OFF, _REM)],
                        out_hbm.at[pl.ds(c * _N + _REM_OFF, _REM)])


_sc_scatter = pl.kernel(
    _sc_scatter_body,
    out_type=jax.ShapeDtypeStruct((_NC * _N, _D), jnp.float32),
    mesh=plsc.VectorSubcoreMesh(core_axis_name="c", subcore_axis_name="s"),
    scratch_types=[
        pltpu.VMEM((_CPB, 2, _CH), jnp.int32),
        pltpu.VMEM((_CPB, 2, _CH), jnp.int32),
        pltpu.VMEM((2, _CH, _D), jnp.float32),
        pltpu.VMEM_SHARED((_ACC_ROWS, _D), jnp.float32),
        pltpu.SemaphoreType.DMA,
        pltpu.SemaphoreType.DMA,
    ],
)


def _mlp_body(h_ref, p_ref, x_ref, w1t_ref, b1_ref, g1_ref, be1_ref,
              w2t_ref, b2_ref, g2_ref, be2_ref, o_ref):
    z = h_ref[...] + p_ref[0:_N, :] + p_ref[_N:2 * _N, :]
    t = jnp.dot(z, w1t_ref[...], preferred_element_type=jnp.float32)
    t = t + b1_ref[...]
    m = jnp.mean(t, axis=0, keepdims=True)
    v = jnp.mean((t - m) * (t - m), axis=0, keepdims=True)
    t = (t - m) / jnp.sqrt(v + 1e-5) * g1_ref[...] + be1_ref[...]
    t = jnp.maximum(t, 0.0)
    u = jnp.dot(t, w2t_ref[...], preferred_element_type=jnp.float32)
    u = u + b2_ref[...]
    m2 = jnp.mean(u, axis=0, keepdims=True)
    v2 = jnp.mean((u - m2) * (u - m2), axis=0, keepdims=True)
    u = (u - m2) / jnp.sqrt(v2 + 1e-5) * g2_ref[...] + be2_ref[...]
    o_ref[...] = jnp.maximum(u, 0.0) + x_ref[...]


_mlp_call = pl.pallas_call(
    _mlp_body,
    out_shape=jax.ShapeDtypeStruct((_N, _D), jnp.float32),
)


def kernel(x, edge_index,
           l0_W1, l0_b1, l0_g1, l0_be1, l0_W2, l0_b2, l0_g2, l0_be2,
           l1_W1, l1_b1, l1_g1, l1_be1, l1_W2, l1_b2, l1_g2, l1_be2):
    pad = _EPAD - _E
    src = jnp.concatenate(
        [edge_index[0].astype(jnp.int32), jnp.zeros((pad,), jnp.int32)])
    dst = jnp.concatenate(
        [edge_index[1].astype(jnp.int32), jnp.full((pad,), _N, jnp.int32)])
    ei = jnp.stack([src.reshape(_NW, _NB, _CPB, _CH),
                    dst.reshape(_NW, _NB, _CPB, _CH)], axis=3)
    zero = jnp.zeros((_N, _D), jnp.float32)

    def layer(h, W1, b1, g1, be1, W2, b2, g2, be2):
        p = _sc_scatter(h, ei, zero)
        return _mlp_call(h, p, x,
                         W1.T, b1.reshape(1, _D), g1.reshape(1, _D),
                         be1.reshape(1, _D),
                         W2.T, b2.reshape(1, _D), g2.reshape(1, _D),
                         be2.reshape(1, _D))

    h = layer(x, l0_W1, l0_b1, l0_g1, l0_be1, l0_W2, l0_b2, l0_g2, l0_be2)
    return layer(h, l1_W1, l1_b1, l1_g1, l1_be1, l1_W2, l1_b2, l1_g2, l1_be2)


# speedup vs baseline: 1.1989x; 1.1989x over previous
import jax
import jax.numpy as jnp
from jax import lax
from jax.experimental import pallas as pl
from jax.experimental.pallas import tpu as pltpu
from jax.experimental.pallas import tpu_sc as plsc

_N, _E, _D = 10000, 320000, 128
_NC, _NS = 2, 16
_NW = _NC * _NS
_CH = 128
_CPB = 16
_NB = 5
_EPT = _NB * _CPB * _CH
_EPAD = _NW * _EPT
_ACC_ROWS = _N + 16
_STRIPE = 624
_REM_OFF = _STRIPE * _NS
_REM = _N - _REM_OFF


def _sc_scatter_body(x_hbm, ei_hbm, zero_hbm, out_hbm,
                     blk_a, blk_b, rows, acc_sh, gsem, ssem):
    c = lax.axis_index("c")
    s = lax.axis_index("s")
    wid = c * _NS + s

    pltpu.sync_copy(zero_hbm.at[pl.ds(s * _STRIPE, _STRIPE)],
                    acc_sh.at[pl.ds(s * _STRIPE, _STRIPE)])

    @pl.when(s == _NS - 1)
    def _():
        pltpu.sync_copy(zero_hbm.at[pl.ds(_REM_OFF, _REM)],
                        acc_sh.at[pl.ds(_REM_OFF, _REM)])

    pltpu.sync_copy(ei_hbm.at[wid, 0], blk_a)
    pltpu.async_copy(ei_hbm.at[wid, 1], blk_b, ssem)
    pltpu.async_copy(x_hbm.at[blk_a.at[0, 0]], rows.at[0], gsem)
    plsc.subcore_barrier()

    def wait_gather(buf):
        pltpu.make_async_copy(x_hbm.at[blk_a.at[0, 0]], buf, gsem).wait()

    def process_block(blk, nxt):
        def pair(i, carry):
            c0 = 2 * i
            c1 = c0 + 1
            wait_gather(rows.at[0])
            pltpu.async_copy(x_hbm.at[blk.at[c1, 0]], rows.at[1], gsem)
            pltpu.sync_copy(rows.at[0], acc_sh.at[blk.at[c0, 1]], add=True)
            wait_gather(rows.at[1])
            pltpu.async_copy(x_hbm.at[blk.at[c0 + 2, 0]], rows.at[0], gsem)
            pltpu.sync_copy(rows.at[1], acc_sh.at[blk.at[c1, 1]], add=True)
            return carry

        lax.fori_loop(0, _CPB // 2 - 1, pair, 0)
        c0 = _CPB - 2
        c1 = _CPB - 1
        wait_gather(rows.at[0])
        pltpu.async_copy(x_hbm.at[blk.at[c1, 0]], rows.at[1], gsem)
        pltpu.sync_copy(rows.at[0], acc_sh.at[blk.at[c0, 1]], add=True)
        wait_gather(rows.at[1])
        if nxt is not None:
            pltpu.make_async_copy(ei_hbm.at[0, 0], nxt, ssem).wait()
            pltpu.async_copy(x_hbm.at[nxt.at[0, 0]], rows.at[0], gsem)
        pltpu.sync_copy(rows.at[1], acc_sh.at[blk.at[c1, 1]], add=True)

    for b in range(_NB):
        blk, nxt = (blk_a, blk_b) if b % 2 == 0 else (blk_b, blk_a)
        process_block(blk, nxt if b < _NB - 1 else None)
        if b + 2 < _NB:
            pltpu.async_copy(ei_hbm.at[wid, b + 2], blk, ssem)

    plsc.subcore_barrier()
    pltpu.sync_copy(acc_sh.at[pl.ds(s * _STRIPE, _STRIPE)],
                    out_hbm.at[pl.ds(c * _N + s * _STRIPE, _STRIPE)])

    @pl.when(s == _NS - 1)
    def _():
        pltpu.sync_copy(acc_sh.at[pl.ds(_REM_OFF, _REM)],
                        out_hbm.at[pl.ds(c * _N + _REM_OFF, _REM)])


_sc_scatter = pl.kernel(
    _sc_scatter_body,
    out_type=jax.ShapeDtypeStruct((_NC * _N, _D), jnp.float32),
    mesh=plsc.VectorSubcoreMesh(core_axis_name="c", subcore_axis_name="s"),
    scratch_types=[
        pltpu.VMEM((_CPB, 2, _CH), jnp.int32),
        pltpu.VMEM((_CPB, 2, _CH), jnp.int32),
        pltpu.VMEM((2, _CH, _D), jnp.float32),
        pltpu.VMEM_SHARED((_ACC_ROWS, _D), jnp.float32),
        pltpu.SemaphoreType.DMA,
        pltpu.SemaphoreType.DMA,
    ],
)


def _mlp_body(h_ref, p_ref, x_ref, w1t_ref, b1_ref, g1_ref, be1_ref,
              w2t_ref, b2_ref, g2_ref, be2_ref, o_ref):
    z = h_ref[...] + p_ref[0:_N, :] + p_ref[_N:2 * _N, :]
    t = jnp.dot(z, w1t_ref[...], preferred_element_type=jnp.float32)
    t = t + b1_ref[...]
    m = jnp.mean(t, axis=0, keepdims=True)
    v = jnp.mean((t - m) * (t - m), axis=0, keepdims=True)
    t = (t - m) / jnp.sqrt(v + 1e-5) * g1_ref[...] + be1_ref[...]
    t = jnp.maximum(t, 0.0)
    u = jnp.dot(t, w2t_ref[...], preferred_element_type=jnp.float32)
    u = u + b2_ref[...]
    m2 = jnp.mean(u, axis=0, keepdims=True)
    v2 = jnp.mean((u - m2) * (u - m2), axis=0, keepdims=True)
    u = (u - m2) / jnp.sqrt(v2 + 1e-5) * g2_ref[...] + be2_ref[...]
    o_ref[...] = jnp.maximum(u, 0.0) + x_ref[...]


_mlp_call = pl.pallas_call(
    _mlp_body,
    out_shape=jax.ShapeDtypeStruct((_N, _D), jnp.float32),
)


def kernel(x, edge_index,
           l0_W1, l0_b1, l0_g1, l0_be1, l0_W2, l0_b2, l0_g2, l0_be2,
           l1_W1, l1_b1, l1_g1, l1_be1, l1_W2, l1_b2, l1_g2, l1_be2):
    rpt = _E // _NW
    pad_pt = _EPT - rpt
    src = jnp.pad(edge_index[0].astype(jnp.int32).reshape(_NW, rpt),
                  ((0, 0), (0, pad_pt)))
    dummy = _N + (jnp.arange(pad_pt, dtype=jnp.int32) % 16)
    dst = jnp.concatenate(
        [edge_index[1].astype(jnp.int32).reshape(_NW, rpt),
         jnp.broadcast_to(dummy, (_NW, pad_pt))], axis=1)
    ei = jnp.stack([src.reshape(_NW, _NB, _CPB, _CH),
                    dst.reshape(_NW, _NB, _CPB, _CH)], axis=3)
    zero = jnp.zeros((_N, _D), jnp.float32)

    def layer(h, W1, b1, g1, be1, W2, b2, g2, be2):
        p = _sc_scatter(h, ei, zero)
        return _mlp_call(h, p, x,
                         W1.T, b1.reshape(1, _D), g1.reshape(1, _D),
                         be1.reshape(1, _D),
                         W2.T, b2.reshape(1, _D), g2.reshape(1, _D),
                         be2.reshape(1, _D))

    h = layer(x, l0_W1, l0_b1, l0_g1, l0_be1, l0_W2, l0_b2, l0_g2, l0_be2)
    return layer(h, l1_W1, l1_b1, l1_g1, l1_be1, l1_W2, l1_b2, l1_g2, l1_be2)


# speedup vs baseline: 3.2234x; 2.6885x over previous
import jax
import jax.numpy as jnp
from jax import lax
from jax.experimental import pallas as pl
from jax.experimental.pallas import tpu as pltpu
from jax.experimental.pallas import tpu_sc as plsc

_N, _E, _D = 10000, 320000, 128
_NC, _NS = 2, 16
_NW = _NC * _NS
_CH = 125
_CPB = 16
_NB = 5
_EPT = _NB * _CPB * _CH
_ACC_ROWS = _N
_STRIPE = 624
_REM_OFF = _STRIPE * _NS
_REM = _N - _REM_OFF


def _sc_scatter_body(x_hbm, ei_hbm, zero_hbm, out_hbm,
                     blk_a, blk_b, rows, acc_sh, gsem, ssem):
    c = lax.axis_index("c")
    s = lax.axis_index("s")
    wid = c * _NS + s

    pltpu.sync_copy(zero_hbm.at[pl.ds(s * _STRIPE, _STRIPE)],
                    acc_sh.at[pl.ds(s * _STRIPE, _STRIPE)])

    @pl.when(s == _NS - 1)
    def _():
        pltpu.sync_copy(zero_hbm.at[pl.ds(_REM_OFF, _REM)],
                        acc_sh.at[pl.ds(_REM_OFF, _REM)])

    pltpu.sync_copy(ei_hbm.at[wid, 0], blk_a)
    pltpu.async_copy(ei_hbm.at[wid, 1], blk_b, ssem)
    pltpu.async_copy(x_hbm.at[blk_a.at[0, 0]], rows.at[0], gsem)
    plsc.subcore_barrier()

    def wait_gather(buf):
        pltpu.make_async_copy(x_hbm.at[blk_a.at[0, 0]], buf, gsem).wait()

    def process_block(blk, nxt):
        def pair(i, carry):
            c0 = 2 * i
            c1 = c0 + 1
            wait_gather(rows.at[0])
            pltpu.async_copy(x_hbm.at[blk.at[c1, 0]], rows.at[1], gsem)
            pltpu.sync_copy(rows.at[0], acc_sh.at[blk.at[c0, 1]], add=True)
            wait_gather(rows.at[1])
            pltpu.async_copy(x_hbm.at[blk.at[c0 + 2, 0]], rows.at[0], gsem)
            pltpu.sync_copy(rows.at[1], acc_sh.at[blk.at[c1, 1]], add=True)
            return carry

        lax.fori_loop(0, _CPB // 2 - 1, pair, 0)
        c0 = _CPB - 2
        c1 = _CPB - 1
        wait_gather(rows.at[0])
        pltpu.async_copy(x_hbm.at[blk.at[c1, 0]], rows.at[1], gsem)
        pltpu.sync_copy(rows.at[0], acc_sh.at[blk.at[c0, 1]], add=True)
        wait_gather(rows.at[1])
        if nxt is not None:
            pltpu.make_async_copy(ei_hbm.at[0, 0], nxt, ssem).wait()
            pltpu.async_copy(x_hbm.at[nxt.at[0, 0]], rows.at[0], gsem)
        pltpu.sync_copy(rows.at[1], acc_sh.at[blk.at[c1, 1]], add=True)

    for b in range(_NB):
        blk, nxt = (blk_a, blk_b) if b % 2 == 0 else (blk_b, blk_a)
        process_block(blk, nxt if b < _NB - 1 else None)
        if b + 2 < _NB:
            pltpu.async_copy(ei_hbm.at[wid, b + 2], blk, ssem)

    plsc.subcore_barrier()
    pltpu.sync_copy(acc_sh.at[pl.ds(s * _STRIPE, _STRIPE)],
                    out_hbm.at[pl.ds(c * _N + s * _STRIPE, _STRIPE)])

    @pl.when(s == _NS - 1)
    def _():
        pltpu.sync_copy(acc_sh.at[pl.ds(_REM_OFF, _REM)],
                        out_hbm.at[pl.ds(c * _N + _REM_OFF, _REM)])


_sc_scatter = pl.kernel(
    _sc_scatter_body,
    out_type=jax.ShapeDtypeStruct((_NC * _N, _D), jnp.float32),
    mesh=plsc.VectorSubcoreMesh(core_axis_name="c", subcore_axis_name="s"),
    scratch_types=[
        pltpu.VMEM((_CPB, 2, _CH), jnp.int32),
        pltpu.VMEM((_CPB, 2, _CH), jnp.int32),
        pltpu.VMEM((2, _CH, _D), jnp.float32),
        pltpu.VMEM_SHARED((_ACC_ROWS, _D), jnp.float32),
        pltpu.SemaphoreType.DMA,
        pltpu.SemaphoreType.DMA,
    ],
)


def _mlp_body(h_ref, p_ref, x_ref, w1t_ref, b1_ref, g1_ref, be1_ref,
              w2t_ref, b2_ref, g2_ref, be2_ref, o_ref):
    z = h_ref[...] + p_ref[0:_N, :] + p_ref[_N:2 * _N, :]
    t = jnp.dot(z, w1t_ref[...], preferred_element_type=jnp.float32)
    t = t + b1_ref[...]
    m = jnp.mean(t, axis=0, keepdims=True)
    v = jnp.mean((t - m) * (t - m), axis=0, keepdims=True)
    t = (t - m) / jnp.sqrt(v + 1e-5) * g1_ref[...] + be1_ref[...]
    t = jnp.maximum(t, 0.0)
    u = jnp.dot(t, w2t_ref[...], preferred_element_type=jnp.float32)
    u = u + b2_ref[...]
    m2 = jnp.mean(u, axis=0, keepdims=True)
    v2 = jnp.mean((u - m2) * (u - m2), axis=0, keepdims=True)
    u = (u - m2) / jnp.sqrt(v2 + 1e-5) * g2_ref[...] + be2_ref[...]
    o_ref[...] = jnp.maximum(u, 0.0) + x_ref[...]


_mlp_call = pl.pallas_call(
    _mlp_body,
    out_shape=jax.ShapeDtypeStruct((_N, _D), jnp.float32),
)


def kernel(x, edge_index,
           l0_W1, l0_b1, l0_g1, l0_be1, l0_W2, l0_b2, l0_g2, l0_be2,
           l1_W1, l1_b1, l1_g1, l1_be1, l1_W2, l1_b2, l1_g2, l1_be2):
    src = edge_index[0].astype(jnp.int32)
    dst = edge_index[1].astype(jnp.int32)
    ei = jnp.stack([src.reshape(_NW, _NB, _CPB, _CH),
                    dst.reshape(_NW, _NB, _CPB, _CH)], axis=3)
    zero = jnp.zeros((_N, _D), jnp.float32)

    def layer(h, W1, b1, g1, be1, W2, b2, g2, be2):
        p = _sc_scatter(h, ei, zero)
        return _mlp_call(h, p, x,
                         W1.T, b1.reshape(1, _D), g1.reshape(1, _D),
                         be1.reshape(1, _D),
                         W2.T, b2.reshape(1, _D), g2.reshape(1, _D),
                         be2.reshape(1, _D))

    h = layer(x, l0_W1, l0_b1, l0_g1, l0_be1, l0_W2, l0_b2, l0_g2, l0_be2)
    return layer(h, l1_W1, l1_b1, l1_g1, l1_be1, l1_W2, l1_b2, l1_g2, l1_be2)


# speedup vs baseline: 3.2273x; 1.0012x over previous
import jax
import jax.numpy as jnp
from jax import lax
from jax.experimental import pallas as pl
from jax.experimental.pallas import tpu as pltpu
from jax.experimental.pallas import tpu_sc as plsc

_N, _E, _D = 10000, 320000, 128
_NC, _NS = 2, 16
_NW = _NC * _NS
_CH = 125
_CPB = 16
_NB = 5
_EPT = _NB * _CPB * _CH
_ACC_ROWS = _N
_STRIPE = 624
_REM_OFF = _STRIPE * _NS
_REM = _N - _REM_OFF


def _sc_scatter_body(x_hbm, ei_hbm, zero_hbm, out_hbm,
                     blk_a, blk_b, rows, acc_sh, g0, g1, c0s, c1s, ssem):
    c = lax.axis_index("c")
    s = lax.axis_index("s")
    wid = c * _NS + s

    pltpu.sync_copy(zero_hbm.at[pl.ds(s * _STRIPE, _STRIPE)],
                    acc_sh.at[pl.ds(s * _STRIPE, _STRIPE)])

    @pl.when(s == _NS - 1)
    def _():
        pltpu.sync_copy(zero_hbm.at[pl.ds(_REM_OFF, _REM)],
                        acc_sh.at[pl.ds(_REM_OFF, _REM)])

    def g_issue(idx_row, buf, sem):
        pltpu.async_copy(x_hbm.at[idx_row], buf, sem)

    def g_wait(buf, sem):
        pltpu.make_async_copy(x_hbm.at[blk_a.at[0, 0]], buf, sem).wait()

    def s_issue(buf, idx_row, sem):
        pltpu.async_copy(buf, acc_sh.at[idx_row], sem, add=True)

    def s_wait(buf, sem):
        pltpu.make_async_copy(buf, acc_sh.at[blk_a.at[0, 1]], sem).wait()

    pltpu.sync_copy(ei_hbm.at[wid, 0], blk_a)
    pltpu.async_copy(ei_hbm.at[wid, 1], blk_b, ssem)
    g_issue(blk_a.at[0, 0], rows.at[0], g0)
    plsc.subcore_barrier()

    def steady_pair(blk, j0):
        g_wait(rows.at[0], g0)
        s_issue(rows.at[0], blk.at[j0, 1], c0s)
        s_wait(rows.at[1], c1s)
        g_issue(blk.at[j0 + 1, 0], rows.at[1], g1)
        g_wait(rows.at[1], g1)
        s_issue(rows.at[1], blk.at[j0 + 1, 1], c1s)
        s_wait(rows.at[0], c0s)
        g_issue(blk.at[j0 + 2, 0], rows.at[0], g0)

    def process_block(b, blk, other):
        g_wait(rows.at[0], g0)
        s_issue(rows.at[0], blk.at[0, 1], c0s)
        if b > 0:
            s_wait(rows.at[1], c1s)
            if b + 1 < _NB:
                pltpu.async_copy(ei_hbm.at[wid, b + 1], other, ssem)
        g_issue(blk.at[1, 0], rows.at[1], g1)
        g_wait(rows.at[1], g1)
        s_issue(rows.at[1], blk.at[1, 1], c1s)
        s_wait(rows.at[0], c0s)
        g_issue(blk.at[2, 0], rows.at[0], g0)

        def pair(i, carry):
            steady_pair(blk, 2 * i)
            return carry

        lax.fori_loop(1, _CPB // 2 - 1, pair, 0)

        j0 = _CPB - 2
        g_wait(rows.at[0], g0)
        s_issue(rows.at[0], blk.at[j0, 1], c0s)
        s_wait(rows.at[1], c1s)
        g_issue(blk.at[j0 + 1, 0], rows.at[1], g1)
        g_wait(rows.at[1], g1)
        s_issue(rows.at[1], blk.at[j0 + 1, 1], c1s)
        s_wait(rows.at[0], c0s)
        if b + 1 < _NB:
            pltpu.make_async_copy(ei_hbm.at[0, 0], other, ssem).wait()
            g_issue(other.at[0, 0], rows.at[0], g0)
        else:
            s_wait(rows.at[1], c1s)

    for b in range(_NB):
        blk, other = (blk_a, blk_b) if b % 2 == 0 else (blk_b, blk_a)
        process_block(b, blk, other)

    plsc.subcore_barrier()
    pltpu.sync_copy(acc_sh.at[pl.ds(s * _STRIPE, _STRIPE)],
                    out_hbm.at[pl.ds(c * _N + s * _STRIPE, _STRIPE)])

    @pl.when(s == _NS - 1)
    def _():
        pltpu.sync_copy(acc_sh.at[pl.ds(_REM_OFF, _REM)],
                        out_hbm.at[pl.ds(c * _N + _REM_OFF, _REM)])


_sc_scatter = pl.kernel(
    _sc_scatter_body,
    out_type=jax.ShapeDtypeStruct((_NC * _N, _D), jnp.float32),
    mesh=plsc.VectorSubcoreMesh(core_axis_name="c", subcore_axis_name="s"),
    scratch_types=[
        pltpu.VMEM((_CPB, 2, _CH), jnp.int32),
        pltpu.VMEM((_CPB, 2, _CH), jnp.int32),
        pltpu.VMEM((2, _CH, _D), jnp.float32),
        pltpu.VMEM_SHARED((_ACC_ROWS, _D), jnp.float32),
        pltpu.SemaphoreType.DMA,
        pltpu.SemaphoreType.DMA,
        pltpu.SemaphoreType.DMA,
        pltpu.SemaphoreType.DMA,
        pltpu.SemaphoreType.DMA,
    ],
)


def _mlp_body(h_ref, p_ref, x_ref, w1t_ref, b1_ref, g1_ref, be1_ref,
              w2t_ref, b2_ref, g2_ref, be2_ref, o_ref):
    z = h_ref[...] + p_ref[0:_N, :] + p_ref[_N:2 * _N, :]
    t = jnp.dot(z, w1t_ref[...], preferred_element_type=jnp.float32)
    t = t + b1_ref[...]
    m = jnp.mean(t, axis=0, keepdims=True)
    v = jnp.mean((t - m) * (t - m), axis=0, keepdims=True)
    t = (t - m) / jnp.sqrt(v + 1e-5) * g1_ref[...] + be1_ref[...]
    t = jnp.maximum(t, 0.0)
    u = jnp.dot(t, w2t_ref[...], preferred_element_type=jnp.float32)
    u = u + b2_ref[...]
    m2 = jnp.mean(u, axis=0, keepdims=True)
    v2 = jnp.mean((u - m2) * (u - m2), axis=0, keepdims=True)
    u = (u - m2) / jnp.sqrt(v2 + 1e-5) * g2_ref[...] + be2_ref[...]
    o_ref[...] = jnp.maximum(u, 0.0) + x_ref[...]


_mlp_call = pl.pallas_call(
    _mlp_body,
    out_shape=jax.ShapeDtypeStruct((_N, _D), jnp.float32),
)


def kernel(x, edge_index,
           l0_W1, l0_b1, l0_g1, l0_be1, l0_W2, l0_b2, l0_g2, l0_be2,
           l1_W1, l1_b1, l1_g1, l1_be1, l1_W2, l1_b2, l1_g2, l1_be2):
    src = edge_index[0].astype(jnp.int32)
    dst = edge_index[1].astype(jnp.int32)
    ei = jnp.stack([src.reshape(_NW, _NB, _CPB, _CH),
                    dst.reshape(_NW, _NB, _CPB, _CH)], axis=3)
    zero = jnp.zeros((_N, _D), jnp.float32)

    def layer(h, W1, b1, g1, be1, W2, b2, g2, be2):
        p = _sc_scatter(h, ei, zero)
        return _mlp_call(h, p, x,
                         W1.T, b1.reshape(1, _D), g1.reshape(1, _D),
                         be1.reshape(1, _D),
                         W2.T, b2.reshape(1, _D), g2.reshape(1, _D),
                         be2.reshape(1, _D))

    h = layer(x, l0_W1, l0_b1, l0_g1, l0_be1, l0_W2, l0_b2, l0_g2, l0_be2)
    return layer(h, l1_W1, l1_b1, l1_g1, l1_be1, l1_W2, l1_b2, l1_g2, l1_be2)


# speedup vs baseline: 3.4987x; 1.0841x over previous
import functools

import jax
import jax.numpy as jnp
from jax import lax
from jax.experimental import pallas as pl
from jax.experimental.pallas import tpu as pltpu
from jax.experimental.pallas import tpu_sc as plsc

_N, _E, _D = 10000, 320000, 128
_NC, _NS = 2, 16
_NW = _NC * _NS
_CH = 125
_CPB = 16
_NB = 5
_STRIPE = 624
_REM_OFF = _STRIPE * _NS
_REM = _N - _REM_OFF
_ZCH = 104


def _sc_scatter_body(x_hbm, ei_hbm, out_hbm,
                     sa, sb, da, db, rows, acc_sh, g0, g1, c0s, c1s, ssem):
    c = lax.axis_index("c")
    s = lax.axis_index("s")
    wid = c * _NS + s

    def zrow(r, carry):
        for j in range(_D // 16):
            rows[0, r, pl.ds(j * 16, 16)] = jnp.zeros((16,), jnp.float32)
        return carry

    lax.fori_loop(0, _ZCH, zrow, 0)
    for k in range(_STRIPE // _ZCH):
        pltpu.sync_copy(rows.at[0].at[pl.ds(0, _ZCH)],
                        acc_sh.at[pl.ds(s * _STRIPE + k * _ZCH, _ZCH)])

    @pl.when(s == _NS - 1)
    def _():
        pltpu.sync_copy(rows.at[0].at[pl.ds(0, _REM)],
                        acc_sh.at[pl.ds(_REM_OFF, _REM)])

    def g_issue(idx_row, buf, sem):
        pltpu.async_copy(x_hbm.at[idx_row], buf, sem)

    def g_wait(buf, sem):
        pltpu.make_async_copy(x_hbm.at[sa.at[0]], buf, sem).wait()

    def s_issue(buf, idx_row, sem):
        pltpu.async_copy(buf, acc_sh.at[idx_row], sem, add=True)

    def s_wait(buf, sem):
        pltpu.make_async_copy(buf, acc_sh.at[da.at[0]], sem).wait()

    def stage(b, sblk, dblk):
        pltpu.async_copy(ei_hbm.at[0, wid, b], sblk, ssem)
        pltpu.async_copy(ei_hbm.at[1, wid, b], dblk, ssem)

    def stage_wait(sblk, dblk):
        pltpu.make_async_copy(ei_hbm.at[0, 0, 0], sblk, ssem).wait()
        pltpu.make_async_copy(ei_hbm.at[1, 0, 0], dblk, ssem).wait()

    stage(0, sa, da)
    stage_wait(sa, da)
    stage(1, sb, db)
    g_issue(sa.at[0], rows.at[0], g0)
    plsc.subcore_barrier()

    def steady_pair(sblk, dblk, j0):
        g_wait(rows.at[0], g0)
        s_issue(rows.at[0], dblk.at[j0], c0s)
        s_wait(rows.at[1], c1s)
        g_issue(sblk.at[j0 + 1], rows.at[1], g1)
        g_wait(rows.at[1], g1)
        s_issue(rows.at[1], dblk.at[j0 + 1], c1s)
        s_wait(rows.at[0], c0s)
        g_issue(sblk.at[j0 + 2], rows.at[0], g0)

    def process_block(b, sblk, dblk, so, do_):
        g_wait(rows.at[0], g0)
        s_issue(rows.at[0], dblk.at[0], c0s)
        if b > 0:
            s_wait(rows.at[1], c1s)
            if b + 1 < _NB:
                stage(b + 1, so, do_)
        g_issue(sblk.at[1], rows.at[1], g1)
        g_wait(rows.at[1], g1)
        s_issue(rows.at[1], dblk.at[1], c1s)
        s_wait(rows.at[0], c0s)
        g_issue(sblk.at[2], rows.at[0], g0)

        def pair(i, carry):
            steady_pair(sblk, dblk, 2 * i)
            return carry

        lax.fori_loop(1, _CPB // 2 - 1, pair, 0)

        j0 = _CPB - 2
        g_wait(rows.at[0], g0)
        s_issue(rows.at[0], dblk.at[j0], c0s)
        s_wait(rows.at[1], c1s)
        g_issue(sblk.at[j0 + 1], rows.at[1], g1)
        g_wait(rows.at[1], g1)
        s_issue(rows.at[1], dblk.at[j0 + 1], c1s)
        s_wait(rows.at[0], c0s)
        if b + 1 < _NB:
            stage_wait(so, do_)
            g_issue(so.at[0], rows.at[0], g0)
        else:
            s_wait(rows.at[1], c1s)

    for b in range(_NB):
        sblk, dblk, so, do_ = (sa, da, sb, db) if b % 2 == 0 else (sb, db, sa, da)
        process_block(b, sblk, dblk, so, do_)

    plsc.subcore_barrier()
    pltpu.sync_copy(acc_sh.at[pl.ds(s * _STRIPE, _STRIPE)],
                    out_hbm.at[pl.ds(c * _N + s * _STRIPE, _STRIPE)])

    @pl.when(s == _NS - 1)
    def _():
        pltpu.sync_copy(acc_sh.at[pl.ds(_REM_OFF, _REM)],
                        out_hbm.at[pl.ds(c * _N + _REM_OFF, _REM)])


_sc_scatter = pl.kernel(
    _sc_scatter_body,
    out_type=jax.ShapeDtypeStruct((_NC * _N, _D), jnp.float32),
    mesh=plsc.VectorSubcoreMesh(core_axis_name="c", subcore_axis_name="s"),
    scratch_types=[
        pltpu.VMEM((_CPB, _CH), jnp.int32),
        pltpu.VMEM((_CPB, _CH), jnp.int32),
        pltpu.VMEM((_CPB, _CH), jnp.int32),
        pltpu.VMEM((_CPB, _CH), jnp.int32),
        pltpu.VMEM((2, _CH, _D), jnp.float32),
        pltpu.VMEM_SHARED((_N, _D), jnp.float32),
        pltpu.SemaphoreType.DMA,
        pltpu.SemaphoreType.DMA,
        pltpu.SemaphoreType.DMA,
        pltpu.SemaphoreType.DMA,
        pltpu.SemaphoreType.DMA,
    ],
)


def _mlp_core(z, x_res, w1t_ref, b1_ref, g1_ref, be1_ref,
              w2t_ref, b2_ref, g2_ref, be2_ref):
    t = jnp.dot(z, w1t_ref[...], preferred_element_type=jnp.float32)
    t = t + b1_ref[...]
    m = jnp.mean(t, axis=0, keepdims=True)
    v = jnp.mean((t - m) * (t - m), axis=0, keepdims=True)
    t = (t - m) / jnp.sqrt(v + 1e-5) * g1_ref[...] + be1_ref[...]
    t = jnp.maximum(t, 0.0)
    u = jnp.dot(t, w2t_ref[...], preferred_element_type=jnp.float32)
    u = u + b2_ref[...]
    m2 = jnp.mean(u, axis=0, keepdims=True)
    v2 = jnp.mean((u - m2) * (u - m2), axis=0, keepdims=True)
    u = (u - m2) / jnp.sqrt(v2 + 1e-5) * g2_ref[...] + be2_ref[...]
    return jnp.maximum(u, 0.0) + x_res


def _mlp0_body(p_ref, x_ref, w1t_ref, b1_ref, g1_ref, be1_ref,
               w2t_ref, b2_ref, g2_ref, be2_ref, o_ref):
    x = x_ref[...]
    z = x + p_ref[0:_N, :] + p_ref[_N:2 * _N, :]
    o_ref[...] = _mlp_core(z, x, w1t_ref, b1_ref, g1_ref, be1_ref,
                           w2t_ref, b2_ref, g2_ref, be2_ref)


def _mlp1_body(h_ref, p_ref, x_ref, w1t_ref, b1_ref, g1_ref, be1_ref,
               w2t_ref, b2_ref, g2_ref, be2_ref, o_ref):
    z = h_ref[...] + p_ref[0:_N, :] + p_ref[_N:2 * _N, :]
    o_ref[...] = _mlp_core(z, x_ref[...], w1t_ref, b1_ref, g1_ref, be1_ref,
                           w2t_ref, b2_ref, g2_ref, be2_ref)


_mlp0_call = pl.pallas_call(
    _mlp0_body, out_shape=jax.ShapeDtypeStruct((_N, _D), jnp.float32))
_mlp1_call = pl.pallas_call(
    _mlp1_body, out_shape=jax.ShapeDtypeStruct((_N, _D), jnp.float32))


def kernel(x, edge_index,
           l0_W1, l0_b1, l0_g1, l0_be1, l0_W2, l0_b2, l0_g2, l0_be2,
           l1_W1, l1_b1, l1_g1, l1_be1, l1_W2, l1_b2, l1_g2, l1_be2):
    ei = edge_index.astype(jnp.int32).reshape(2, _NW, _NB, _CPB, _CH)

    def wparams(W1, b1, g1, be1, W2, b2, g2, be2):
        return (W1.T, b1.reshape(1, _D), g1.reshape(1, _D),
                be1.reshape(1, _D),
                W2.T, b2.reshape(1, _D), g2.reshape(1, _D),
                be2.reshape(1, _D))

    p0 = _sc_scatter(x, ei)
    h = _mlp0_call(p0, x, *wparams(l0_W1, l0_b1, l0_g1, l0_be1,
                                   l0_W2, l0_b2, l0_g2, l0_be2))
    p1 = _sc_scatter(h, ei)
    return _mlp1_call(h, p1, x, *wparams(l1_W1, l1_b1, l1_g1, l1_be1,
                                         l1_W2, l1_b2, l1_g2, l1_be2))


# speedup vs baseline: 3.5185x; 1.0057x over previous
import functools

import jax
import jax.numpy as jnp
from jax import lax
from jax.experimental import pallas as pl
from jax.experimental.pallas import tpu as pltpu
from jax.experimental.pallas import tpu_sc as plsc

_N, _E, _D = 10000, 320000, 128
_NC, _NS = 2, 16
_NW = _NC * _NS
_CH = 125
_CPB = 16
_NB = 5
_BLK = _CPB * _CH
_STRIPE = 624
_REM_OFF = _STRIPE * _NS
_REM = _N - _REM_OFF
_ZCH = 104


def _sc_scatter_body(x_hbm, ei_hbm, out_hbm,
                     sa, sb, da, db, rows, acc_sh, g0, g1, c0s, c1s, ssem):
    c = lax.axis_index("c")
    s = lax.axis_index("s")
    wid = c * _NS + s

    def stage(b, sblk, dblk):
        pltpu.async_copy(ei_hbm.at[0, wid, b], sblk, ssem)
        pltpu.async_copy(ei_hbm.at[1, wid, b], dblk, ssem)

    def stage_wait(sblk, dblk):
        pltpu.make_async_copy(ei_hbm.at[0, 0, 0], sblk, ssem).wait()
        pltpu.make_async_copy(ei_hbm.at[1, 0, 0], dblk, ssem).wait()

    def g_issue(idx_row, buf, sem):
        pltpu.async_copy(x_hbm.at[idx_row], buf, sem)

    def g_wait(buf, sem):
        pltpu.make_async_copy(x_hbm.at[sa.at[0]], buf, sem).wait()

    def s_issue(buf, idx_row, sem):
        pltpu.async_copy(buf, acc_sh.at[idx_row], sem, add=True)

    def s_wait(buf, sem):
        pltpu.make_async_copy(buf, acc_sh.at[da.at[0]], sem).wait()

    stage(0, sa, da)
    pltpu.make_async_copy(ei_hbm.at[0, 0, 0], sa, ssem).wait()
    g_issue(sa.at[0], rows.at[0], g0)

    def zrow(r, carry):
        for j in range(_D // 16):
            rows[1, r, pl.ds(j * 16, 16)] = jnp.zeros((16,), jnp.float32)
        return carry

    lax.fori_loop(0, _ZCH, zrow, 0)
    for k in range(_STRIPE // _ZCH):
        pltpu.async_copy(rows.at[1].at[pl.ds(0, _ZCH)],
                         acc_sh.at[pl.ds(s * _STRIPE + k * _ZCH, _ZCH)], c1s)

    @pl.when(s == _NS - 1)
    def _():
        pltpu.async_copy(rows.at[1].at[pl.ds(0, _REM)],
                         acc_sh.at[pl.ds(_REM_OFF, _REM)], c1s)

    pltpu.make_async_copy(ei_hbm.at[1, 0, 0], da, ssem).wait()
    stage(1, sb, db)
    for k in range(_STRIPE // _ZCH):
        pltpu.make_async_copy(rows.at[1].at[pl.ds(0, _ZCH)],
                              acc_sh.at[pl.ds(0, _ZCH)], c1s).wait()

    @pl.when(s == _NS - 1)
    def _():
        pltpu.make_async_copy(rows.at[1].at[pl.ds(0, _REM)],
                              acc_sh.at[pl.ds(0, _REM)], c1s).wait()

    plsc.subcore_barrier()

    def idx_row(blk, j):
        return blk.at[j]

    def steady_pair(sblk, dblk, j0):
        g_wait(rows.at[0], g0)
        s_issue(rows.at[0], idx_row(dblk, j0), c0s)
        s_wait(rows.at[1], c1s)
        g_issue(idx_row(sblk, j0 + 1), rows.at[1], g1)
        g_wait(rows.at[1], g1)
        s_issue(rows.at[1], idx_row(dblk, j0 + 1), c1s)
        s_wait(rows.at[0], c0s)
        g_issue(idx_row(sblk, j0 + 2), rows.at[0], g0)

    def process_block(b, sblk, dblk, so, do_):
        g_wait(rows.at[0], g0)
        s_issue(rows.at[0], idx_row(dblk, 0), c0s)
        if b > 0:
            s_wait(rows.at[1], c1s)
            if b + 1 < _NB:
                stage(b + 1, so, do_)
        g_issue(idx_row(sblk, 1), rows.at[1], g1)
        g_wait(rows.at[1], g1)
        s_issue(rows.at[1], idx_row(dblk, 1), c1s)
        s_wait(rows.at[0], c0s)
        g_issue(idx_row(sblk, 2), rows.at[0], g0)

        def pair(i, carry):
            steady_pair(sblk, dblk, 2 * i)
            return carry

        lax.fori_loop(1, _CPB // 2 - 1, pair, 0)

        j0 = _CPB - 2
        g_wait(rows.at[0], g0)
        s_issue(rows.at[0], idx_row(dblk, j0), c0s)
        s_wait(rows.at[1], c1s)
        g_issue(idx_row(sblk, j0 + 1), rows.at[1], g1)
        g_wait(rows.at[1], g1)
        s_issue(rows.at[1], idx_row(dblk, j0 + 1), c1s)
        s_wait(rows.at[0], c0s)
        if b + 1 < _NB:
            stage_wait(so, do_)
            g_issue(idx_row(so, 0), rows.at[0], g0)
        else:
            s_wait(rows.at[1], c1s)

    for b in range(_NB):
        sblk, dblk, so, do_ = (sa, da, sb, db) if b % 2 == 0 else (sb, db, sa, da)
        process_block(b, sblk, dblk, so, do_)

    plsc.subcore_barrier()
    pltpu.sync_copy(acc_sh.at[pl.ds(s * _STRIPE, _STRIPE)],
                    out_hbm.at[pl.ds(c * _N + s * _STRIPE, _STRIPE)])

    @pl.when(s == _NS - 1)
    def _():
        pltpu.sync_copy(acc_sh.at[pl.ds(_REM_OFF, _REM)],
                        out_hbm.at[pl.ds(c * _N + _REM_OFF, _REM)])


_sc_scatter = pl.kernel(
    _sc_scatter_body,
    out_type=jax.ShapeDtypeStruct((_NC * _N, _D), jnp.float32),
    mesh=plsc.VectorSubcoreMesh(core_axis_name="c", subcore_axis_name="s"),
    scratch_types=[
        pltpu.VMEM((_CPB, _CH), jnp.int32),
        pltpu.VMEM((_CPB, _CH), jnp.int32),
        pltpu.VMEM((_CPB, _CH), jnp.int32),
        pltpu.VMEM((_CPB, _CH), jnp.int32),
        pltpu.VMEM((2, _CH, _D), jnp.float32),
        pltpu.VMEM_SHARED((_N, _D), jnp.float32),
        pltpu.SemaphoreType.DMA,
        pltpu.SemaphoreType.DMA,
        pltpu.SemaphoreType.DMA,
        pltpu.SemaphoreType.DMA,
        pltpu.SemaphoreType.DMA,
    ],
)


def _mlp_core(z, x_res, w1t_ref, b1_ref, g1_ref, be1_ref,
              w2t_ref, b2_ref, g2_ref, be2_ref):
    t = jnp.dot(z, w1t_ref[...], preferred_element_type=jnp.float32)
    t = t + b1_ref[...]
    m = jnp.mean(t, axis=0, keepdims=True)
    v = jnp.mean((t - m) * (t - m), axis=0, keepdims=True)
    t = (t - m) / jnp.sqrt(v + 1e-5) * g1_ref[...] + be1_ref[...]
    t = jnp.maximum(t, 0.0)
    u = jnp.dot(t, w2t_ref[...], preferred_element_type=jnp.float32)
    u = u + b2_ref[...]
    m2 = jnp.mean(u, axis=0, keepdims=True)
    v2 = jnp.mean((u - m2) * (u - m2), axis=0, keepdims=True)
    u = (u - m2) / jnp.sqrt(v2 + 1e-5) * g2_ref[...] + be2_ref[...]
    return jnp.maximum(u, 0.0) + x_res


def _mlp0_body(p_ref, x_ref, w1t_ref, b1_ref, g1_ref, be1_ref,
               w2t_ref, b2_ref, g2_ref, be2_ref, o_ref):
    x = x_ref[...]
    z = x + p_ref[0:_N, :] + p_ref[_N:2 * _N, :]
    o_ref[...] = _mlp_core(z, x, w1t_ref, b1_ref, g1_ref, be1_ref,
                           w2t_ref, b2_ref, g2_ref, be2_ref)


def _mlp1_body(h_ref, p_ref, x_ref, w1t_ref, b1_ref, g1_ref, be1_ref,
               w2t_ref, b2_ref, g2_ref, be2_ref, o_ref):
    z = h_ref[...] + p_ref[0:_N, :] + p_ref[_N:2 * _N, :]
    o_ref[...] = _mlp_core(z, x_ref[...], w1t_ref, b1_ref, g1_ref, be1_ref,
                           w2t_ref, b2_ref, g2_ref, be2_ref)


_mlp0_call = pl.pallas_call(
    _mlp0_body, out_shape=jax.ShapeDtypeStruct((_N, _D), jnp.float32))
_mlp1_call = pl.pallas_call(
    _mlp1_body, out_shape=jax.ShapeDtypeStruct((_N, _D), jnp.float32))


def kernel(x, edge_index,
           l0_W1, l0_b1, l0_g1, l0_be1, l0_W2, l0_b2, l0_g2, l0_be2,
           l1_W1, l1_b1, l1_g1, l1_be1, l1_W2, l1_b2, l1_g2, l1_be2):
    ei = edge_index.astype(jnp.int32).reshape(2, _NW, _NB, _CPB, _CH)

    def wparams(W1, b1, g1, be1, W2, b2, g2, be2):
        return (W1.T, b1.reshape(1, _D), g1.reshape(1, _D),
                be1.reshape(1, _D),
                W2.T, b2.reshape(1, _D), g2.reshape(1, _D),
                be2.reshape(1, _D))

    p0 = _sc_scatter(x, ei)
    h = _mlp0_call(p0, x, *wparams(l0_W1, l0_b1, l0_g1, l0_be1,
                                   l0_W2, l0_b2, l0_g2, l0_be2))
    p1 = _sc_scatter(h, ei)
    return _mlp1_call(h, p1, x, *wparams(l1_W1, l1_b1, l1_g1, l1_be1,
                                         l1_W2, l1_b2, l1_g2, l1_be2))
